# baseline (speedup 1.0000x reference)
import functools

import jax
import jax.numpy as jnp
from jax import lax
from jax.experimental import pallas as pl
from jax.experimental.pallas import tpu as pltpu
from jax.experimental.pallas import tpu_sc as plsc

U = 10000
D = 128
E = 320000
NC = 2
NS = 16
NW = NC * NS
EPT = E // NW
CHUNK = 128
GSIZE = 4
NCHUNK = 80
NGRP = NCHUNK // GSIZE
EPTP = NCHUNK * CHUNK
ROWS_PT = 632
UPAD = NS * ROWS_PT


def _spmm4(tables, dsts, srcs, zrows):
    mesh = plsc.VectorSubcoreMesh(core_axis_name="c", subcore_axis_name="s")
    out_type = jax.ShapeDtypeStruct((4, NC, UPAD, D), jnp.float32)
    scratch = [
        pltpu.VMEM((GSIZE, CHUNK), jnp.int32),
        pltpu.VMEM((GSIZE, CHUNK), jnp.int32),
        pltpu.VMEM((GSIZE, CHUNK), jnp.int32),
        pltpu.VMEM((GSIZE, CHUNK), jnp.int32),
        pltpu.VMEM((CHUNK, D), jnp.float32),
        pltpu.VMEM((CHUNK, D), jnp.float32),
        pltpu.VMEM_SHARED((UPAD, D), jnp.float32),
        pltpu.SemaphoreType.DMA,
        pltpu.SemaphoreType.DMA,
        pltpu.SemaphoreType.DMA,
        pltpu.SemaphoreType.DMA,
        pltpu.SemaphoreType.DMA,
    ]

    @functools.partial(pl.kernel, mesh=mesh, out_type=out_type,
                       scratch_types=scratch)
    def k(zr, t0, t1, t2, t3, d0, d1, d2, d3, s0, s1, s2, s3,
          out, dr0, dr1, sr0, sr1, rows0, rows1, acc,
          gsem0, gsem1, rsem0, rsem1, zsem):
        cid = lax.axis_index("c")
        sid = lax.axis_index("s")
        wid = sid * NC + cid
        rbase = sid * ROWS_PT
        rowbuf = ((rows0, gsem0), (rows1, gsem1))

        for p, (tb, dh, sh) in enumerate(((t0, d0, s0), (t1, d1, s1),
                                          (t2, d2, s2), (t3, d3, s3))):
            tbl = tb if tb.shape == (U, D) else tb.at[p]

            def load_group(g, dr, sr, rsem):
                pltpu.async_copy(dh.at[wid, pl.ds(g * GSIZE, GSIZE)],
                                 dr, rsem)
                pltpu.async_copy(sh.at[wid, pl.ds(g * GSIZE, GSIZE)],
                                 sr, rsem)

            def wait_group(g, dr, sr, rsem):
                pltpu.make_async_copy(
                    dh.at[wid, pl.ds(g * GSIZE, GSIZE)], dr, rsem).wait()
                pltpu.make_async_copy(
                    sh.at[wid, pl.ds(g * GSIZE, GSIZE)], sr, rsem).wait()

            pltpu.async_copy(zr, acc.at[pl.ds(rbase, ROWS_PT)], zsem)
            load_group(0, dr0, sr0, rsem0)
            load_group(1, dr1, sr1, rsem1)
            pltpu.make_async_copy(zr, acc.at[pl.ds(rbase, ROWS_PT)],
                                  zsem).wait()
            wait_group(0, dr0, sr0, rsem0)
            plsc.subcore_barrier()
            pltpu.async_copy(tbl.at[sr0.at[0]], rows0, gsem0)

            def group_body(g, dr, sr, rsem_this, dr_n, sr_n, rsem_next):
                for kk in range(GSIZE):
                    rows_b, gsem_b = rowbuf[kk % 2]
                    rows_o, gsem_o = rowbuf[1 - kk % 2]
                    if kk == GSIZE - 2:
                        @pl.when(g + 1 < NGRP)
                        def _():
                            wait_group(g + 1, dr_n, sr_n, rsem_next)
                    if kk < GSIZE - 1:
                        pltpu.async_copy(tbl.at[sr.at[kk + 1]], rows_o,
                                         gsem_o)
                    else:
                        @pl.when(g + 1 < NGRP)
                        def _():
                            pltpu.async_copy(tbl.at[sr_n.at[0]], rows_o,
                                             gsem_o)
                    pltpu.make_async_copy(tbl.at[sr.at[kk]], rows_b,
                                          gsem_b).wait()
                    pltpu.sync_copy(rows_b, acc.at[dr.at[kk]], add=True)

                @pl.when(g + 2 < NGRP)
                def _():
                    load_group(g + 2, dr, sr, rsem_this)

            def pair(gg, _):
                g0 = 2 * gg
                group_body(g0, dr0, sr0, rsem0, dr1, sr1, rsem1)
                group_body(g0 + 1, dr1, sr1, rsem1, dr0, sr0, rsem0)
                return 0
            lax.fori_loop(0, NGRP // 2, pair, 0)
            plsc.subcore_barrier()

            pltpu.sync_copy(acc.at[pl.ds(rbase, ROWS_PT)],
                            out.at[p, cid, pl.ds(rbase, ROWS_PT)])

    return k(zrows, *tables, *dsts, *srcs)


def _mm4(p, ws, relu):
    blk = 1000

    def body(p_ref, w_ref, o_ref):
        x = p_ref[0, 0] + p_ref[0, 1]
        y = jnp.dot(x, w_ref[0], preferred_element_type=jnp.float32)
        if relu:
            y = jnp.maximum(y, 0.0)
        o_ref[0] = y

    return pl.pallas_call(
        body,
        grid=(4, U // blk),
        in_specs=[pl.BlockSpec((1, NC, blk, D), lambda k, i: (k, 0, i, 0)),
                  pl.BlockSpec((1, D, D), lambda k, i: (k, 0, 0))],
        out_specs=pl.BlockSpec((1, blk, D), lambda k, i: (k, i, 0)),
        out_shape=jax.ShapeDtypeStruct((4, U, D), jnp.float32),
    )(p, ws)


def kernel(source_UV, source_VU, target_UV, target_VU,
           source_UU_adj, source_VV_adj, target_UU_adj, target_VV_adj,
           source_user_table, source_item_table,
           target_user_table, target_item_table,
           s_W_user, s_W_item, s_W_out_u, s_W_out_i,
           t_W_user, t_W_item, t_W_out_u, t_W_out_i):
    npad = EPTP - EPT
    dpad = jnp.broadcast_to(U + (jnp.arange(npad, dtype=jnp.int32)
                                 % (UPAD - U)), (NW, npad))
    spad = jnp.zeros((NW, npad), jnp.int32)

    def e(a):
        a = jnp.asarray(a, jnp.int32).reshape(NW, EPT)
        return jnp.concatenate((a, dpad), axis=1).reshape(NW, NCHUNK, CHUNK)

    def f(a):
        a = jnp.asarray(a, jnp.int32).reshape(NW, EPT)
        return jnp.concatenate((a, spad), axis=1).reshape(NW, NCHUNK, CHUNK)

    zrows = jnp.zeros((ROWS_PT, D), jnp.float32)

    agg = _spmm4(
        (source_item_table, source_user_table,
         target_item_table, target_user_table),
        (e(source_UV[0]), e(source_VU[0]), e(target_UV[0]), e(target_VU[0])),
        (f(source_UV[1]), f(source_VU[1]), f(target_UV[1]), f(target_VU[1])),
        zrows)

    h = _mm4(agg, jnp.stack((s_W_user, s_W_item, t_W_user, t_W_item)), True)

    h2 = _spmm4(
        (h, h, h, h),
        (e(source_UU_adj[0]), e(source_VV_adj[0]),
         e(target_UU_adj[0]), e(target_VV_adj[0])),
        (f(source_UU_adj[1]), f(source_VV_adj[1]),
         f(target_UU_adj[1]), f(target_VV_adj[1])),
        zrows)

    mu = _mm4(h2, jnp.stack((s_W_out_u, s_W_out_i, t_W_out_u, t_W_out_i)),
              False)
    return (mu[0], mu[1], mu[2], mu[3])


# speedup vs baseline: 3.2387x; 3.2387x over previous
import functools

import jax
import jax.numpy as jnp
from jax import lax
from jax.experimental import pallas as pl
from jax.experimental.pallas import tpu as pltpu
from jax.experimental.pallas import tpu_sc as plsc

U = 10000
D = 128
E = 320000
NC = 2
NS = 16
NW = NC * NS
EPT = E // NW
CHUNK = 80
NCHUNK = EPT // CHUNK
ROWS_PT = 632
UPAD = NS * ROWS_PT


def _spmm2(tables, dsts, srcs, zrows):
    mesh = plsc.VectorSubcoreMesh(core_axis_name="c", subcore_axis_name="s")
    out_type = jax.ShapeDtypeStruct((2, NC, UPAD, D), jnp.float32)
    scratch = [
        pltpu.VMEM((NCHUNK, CHUNK), jnp.int32),
        pltpu.VMEM((EPT,), jnp.int32),
        pltpu.VMEM((CHUNK, D), jnp.float32),
        pltpu.VMEM((CHUNK, D), jnp.float32),
        pltpu.VMEM_SHARED((UPAD, D), jnp.float32),
        pltpu.SemaphoreType.DMA,
        pltpu.SemaphoreType.DMA,
        pltpu.SemaphoreType.DMA,
    ]

    @functools.partial(pl.kernel, mesh=mesh, out_type=out_type,
                       scratch_types=scratch)
    def k(zr, t0, t1, d0, d1, s0, s1, out, dst_v, src_v, rows0, rows1, acc,
          sem0, sem1, sem2):
        cid = lax.axis_index("c")
        sid = lax.axis_index("s")
        wid = sid * NC + cid
        rbase = sid * ROWS_PT

        for p, (tb, dh, sh) in enumerate(((t0, d0, s0), (t1, d1, s1))):
            tbl = tb if tb.shape == (U, D) else tb.at[p]
            pltpu.async_copy(zr, acc.at[pl.ds(rbase, ROWS_PT)], sem2)
            pltpu.async_copy(dh.at[wid], dst_v, sem2)
            pltpu.async_copy(sh.at[wid], src_v, sem2)
            pltpu.make_async_copy(zr, acc.at[pl.ds(rbase, ROWS_PT)],
                                  sem2).wait()
            pltpu.make_async_copy(dh.at[wid], dst_v, sem2).wait()
            pltpu.make_async_copy(sh.at[wid], src_v, sem2).wait()
            plsc.subcore_barrier()

            def sidx(c):
                return src_v.at[pl.ds(c * CHUNK, CHUNK)]

            pltpu.async_copy(tbl.at[sidx(0)], rows0, sem0)

            def step(i, _):
                c0 = 2 * i
                c1 = c0 + 1
                pltpu.async_copy(tbl.at[sidx(c1)], rows1, sem1)
                pltpu.make_async_copy(tbl.at[sidx(c0)], rows0, sem0).wait()
                pltpu.sync_copy(rows0, acc.at[dst_v.at[c0]], add=True)
                pltpu.async_copy(tbl.at[sidx(c1 + 1)], rows0, sem0)
                pltpu.make_async_copy(tbl.at[sidx(c1)], rows1, sem1).wait()
                pltpu.sync_copy(rows1, acc.at[dst_v.at[c1]], add=True)
                return 0
            lax.fori_loop(0, (NCHUNK - 1) // 2, step, 0)
            last = NCHUNK - 1
            pltpu.make_async_copy(tbl.at[sidx(last)], rows0, sem0).wait()
            pltpu.sync_copy(rows0, acc.at[dst_v.at[last]], add=True)
            plsc.subcore_barrier()

            pltpu.sync_copy(acc.at[pl.ds(rbase, ROWS_PT)],
                            out.at[p, cid, pl.ds(rbase, ROWS_PT)])

    return k(zrows, *tables, *dsts, *srcs)


def _mm2(p, ws, relu):
    blk = 1000

    def body(p_ref, w_ref, o_ref):
        x = p_ref[0, 0] + p_ref[0, 1]
        y = jnp.dot(x, w_ref[0], preferred_element_type=jnp.float32)
        if relu:
            y = jnp.maximum(y, 0.0)
        o_ref[0] = y

    return pl.pallas_call(
        body,
        grid=(2, U // blk),
        in_specs=[pl.BlockSpec((1, NC, blk, D), lambda k, i: (k, 0, i, 0)),
                  pl.BlockSpec((1, D, D), lambda k, i: (k, 0, 0))],
        out_specs=pl.BlockSpec((1, blk, D), lambda k, i: (k, i, 0)),
        out_shape=jax.ShapeDtypeStruct((2, U, D), jnp.float32),
    )(p, ws)


def kernel(source_UV, source_VU, target_UV, target_VU,
           source_UU_adj, source_VV_adj, target_UU_adj, target_VV_adj,
           source_user_table, source_item_table,
           target_user_table, target_item_table,
           s_W_user, s_W_item, s_W_out_u, s_W_out_i,
           t_W_user, t_W_item, t_W_out_u, t_W_out_i):
    def e(a):
        return jnp.asarray(a, jnp.int32).reshape(NW, NCHUNK, CHUNK)

    def f(a):
        return jnp.asarray(a, jnp.int32).reshape(NW, EPT)

    zrows = jnp.zeros((ROWS_PT, D), jnp.float32)

    agg_s = _spmm2((source_item_table, source_user_table),
                   (e(source_UV[0]), e(source_VU[0])),
                   (f(source_UV[1]), f(source_VU[1])), zrows)
    agg_t = _spmm2((target_item_table, target_user_table),
                   (e(target_UV[0]), e(target_VU[0])),
                   (f(target_UV[1]), f(target_VU[1])), zrows)

    h_s = _mm2(agg_s, jnp.stack((s_W_user, s_W_item)), True)
    h_t = _mm2(agg_t, jnp.stack((t_W_user, t_W_item)), True)

    h2_s = _spmm2((h_s, h_s),
                  (e(source_UU_adj[0]), e(source_VV_adj[0])),
                  (f(source_UU_adj[1]), f(source_VV_adj[1])), zrows)
    h2_t = _spmm2((h_t, h_t),
                  (e(target_UU_adj[0]), e(target_VV_adj[0])),
                  (f(target_UU_adj[1]), f(target_VV_adj[1])), zrows)

    mu_s = _mm2(h2_s, jnp.stack((s_W_out_u, s_W_out_i)), False)
    mu_t = _mm2(h2_t, jnp.stack((t_W_out_u, t_W_out_i)), False)
    return (mu_s[0], mu_s[1], mu_t[0], mu_t[1])
